# Initial kernel scaffold; baseline (speedup 1.0000x reference)
import jax
import jax.numpy as jnp
from jax.experimental import pallas as pl


def kernel(vertex_feature, edge_feature, params, edge_index, rev_h1_index, rev_h2_index):
    raise NotImplementedError("write your pallas kernel here")


# speedup vs baseline: 1.7224x; 1.7224x over previous
import numpy as np
import jax
import jax.numpy as jnp
from jax import lax
from jax.experimental import pallas as pl
from jax.experimental.pallas import tpu as pltpu

G = 500
V = 20
DEG = 16
E_PER = V * DEG
E = G * E_PER
EH = E // 2
EHP = E_PER // 2
M = 128
D_FEAT = 128
EDGE_FDIM = 16
EMB = 128
HID = 128
DEPTH = 3
A1 = 100
A2 = 200
PPER = V * (V - 1) // 2
EH2 = EMB // 2

_iu0, _iu1 = np.triu_indices(V, k=1)
_SAT = jnp.asarray((np.arange(V)[:, None] == _iu0[None, :]).astype(np.float32))
_SBT = jnp.asarray((np.arange(V)[:, None] == _iu1[None, :]).astype(np.float32))

_PNAMES = [
    'W_edge', 'b_edge', 'W_msg', 'b_msg',
    'Wv_a', 'Wv_b', 'b_vert',
    'W_p1', 'b_p1', 'W_p2', 'b_p2',
    'ws_row', 'b_stop',
    'W1h1_0', 'W1h1_1', 'b1h1', 'Wet_h1', 'Web_h1', 'w2h1', 'b2h1',
    'W1h2_0', 'W1h2_1', 'b1h2', 'Wet_h2', 'Web_h2', 'w2h2', 'b2h2',
    'W1r1_0', 'b1r1', 'Wet_r1', 'Web_r1', 'w2r1', 'b2r1',
    'W1r2_0', 'W1r2_1', 'W1r2_2', 'b1r2', 'Wet_r2', 'Web_r2', 'w2r2', 'b2r2',
]


def _prep_params(p):
    d = {}
    d['W_edge'] = p['W_edge']
    d['b_edge'] = p['b_edge'].reshape(1, M)
    d['W_msg'] = p['W_msg']
    d['b_msg'] = p['b_msg'].reshape(1, M)
    d['Wv_a'] = p['W_vert'][:D_FEAT]
    d['Wv_b'] = p['W_vert'][D_FEAT:]
    d['b_vert'] = p['b_vert'].reshape(1, M)
    d['W_p1'] = p['W_p1']
    d['b_p1'] = p['b_p1'].reshape(1, EMB)
    d['W_p2'] = p['W_p2']
    d['b_p2'] = p['b_p2'].reshape(1, EH2)
    d['ws_row'] = p['W_stop'].T.reshape(1, EMB)
    d['b_stop'] = p['b_stop'].reshape(1, 1)
    for name, tag, nsplit in (('h1', 'h1', 2), ('h2', 'h2', 2),
                              ('rh1', 'r1', 1), ('rh2', 'r2', 3)):
        W1 = p['W1_' + name]
        for j in range(nsplit):
            d[f'W1{tag}_{j}'] = W1[j * M:(j + 1) * M]
        d[f'b1{tag}'] = p['b1_' + name].reshape(1, HID)
        We = p['We_' + name]
        d[f'Wet_{tag}'] = We[:EH2]
        d[f'Web_{tag}'] = We[EH2:]
        d[f'w2{tag}'] = p['W2_' + name].T.reshape(1, HID)
        d[f'b2{tag}'] = p['b2_' + name].reshape(1, 1)
    return d


def _dT(a, b):
    return lax.dot_general(a, b, (((0,), (0,)), ((), ())),
                           preferred_element_type=jnp.float32,
                           precision=lax.Precision.HIGHEST)


def _mm(a, b):
    return jnp.dot(a, b, preferred_element_type=jnp.float32,
                   precision=lax.Precision.HIGHEST)


def _body(*refs):
    vf_ref, ef_ref, ei_ref, r1_ref, r2_ref, sat_ref, sbt_ref = refs[:7]
    npar = len(_PNAMES)
    P = {n: refs[7 + i][...] for i, n in enumerate(_PNAMES)}
    out_stop, out_h1, out_h2, out_r1, out_r2 = refs[7 + npar:]
    relu = jax.nn.relu

    ef = ef_ref[0]
    h0 = jnp.tanh(_mm(ef, P['W_edge']) + P['b_edge'])
    h0a = h0[:EHP]
    h0b = h0[EHP:]

    sl = ei_ref[0, 0:1, :]
    dl = ei_ref[0, 1:2, :]
    vio = lax.broadcasted_iota(jnp.int32, (V, EHP), 0)
    PT = (sl == vio).astype(jnp.float32)
    QT = (dl == vio).astype(jnp.float32)

    ha, hb = h0a, h0b
    for _ in range(DEPTH):
        agg = _mm(QT, ha) + _mm(PT, hb)
        ga = _dT(PT, agg)
        gb = _dT(QT, agg)
        ha, hb = (relu(h0a + _mm(ga - hb, P['W_msg']) + P['b_msg']),
                  relu(h0b + _mm(gb - ha, P['W_msg']) + P['b_msg']))
    aggf = _mm(QT, ha) + _mm(PT, hb)

    vf = vf_ref[0]
    vm = relu(_mm(vf, P['Wv_a']) + _mm(aggf, P['Wv_b']) + P['b_vert'])
    pre = _mm(relu(_mm(vm, P['W_p1']) + P['b_p1']), P['W_p2']) + P['b_p2']
    gmean = jnp.mean(pre, axis=0, keepdims=True)
    gmax = jnp.max(pre, axis=0, keepdims=True)

    stopv = (jnp.sum(gmean * P['ws_row'][:, :EH2], axis=1, keepdims=True) +
             jnp.sum(gmax * P['ws_row'][:, EH2:], axis=1, keepdims=True) +
             P['b_stop'][0, 0])
    out_stop[0] = stopv

    def head_c(tag):
        return (_mm(gmean, P[f'Wet_{tag}']) + _mm(gmax, P[f'Web_{tag}']) +
                P[f'b1{tag}'])

    ma = _dT(sat_ref[...], vm)
    mb = _dT(sbt_ref[...], vm)
    hid = relu(_mm(jnp.minimum(ma, mb), P['W1h1_0']) +
               _mm(jnp.maximum(ma, mb), P['W1h1_1']) + head_c('h1'))
    out_h1[0] = (jnp.sum(hid * P['w2h1'], axis=1, keepdims=True) +
                 P['b2h1'][0, 0])

    A = _mm(vm, P['W1h2_0']) + head_c('h2')
    B = _mm(0.5 * (ha + hb), P['W1h2_1'])
    hid3 = relu(A[:, None, :] + B[None, :, :])
    out_h2[0] = (jnp.sum(hid3 * P['w2h2'][None, :, :], axis=2) +
                 P['b2h2'][0, 0])

    vio1 = lax.broadcasted_iota(jnp.int32, (V, A1), 0)
    RT = (r1_ref[0] == vio1).astype(jnp.float32)
    f1 = _dT(RT, vm)
    hid = relu(_mm(f1, P['W1r1_0']) + head_c('r1'))
    out_r1[0] = (jnp.sum(hid * P['w2r1'], axis=1, keepdims=True) +
                 P['b2r1'][0, 0])

    r2 = r2_ref[0]
    vio2 = lax.broadcasted_iota(jnp.int32, (V, A2), 0)
    T0 = (r2[0:1, :] == vio2).astype(jnp.float32)
    T1 = (r2[1:2, :] == vio2).astype(jnp.float32)
    T2 = (r2[2:3, :] == vio2).astype(jnp.float32)
    m0 = _dT(T0, vm)
    m1 = _dT(T1, vm)
    m2 = _dT(T2, vm)
    hid = relu(_mm(m0, P['W1r2_0']) +
               _mm(jnp.minimum(m1, m2), P['W1r2_1']) +
               _mm(jnp.maximum(m1, m2), P['W1r2_2']) + head_c('r2'))
    out_r2[0] = (jnp.sum(hid * P['w2r2'], axis=1, keepdims=True) +
                 P['b2r2'][0, 0])


def kernel(vertex_feature, edge_feature, params, edge_index, rev_h1_index,
           rev_h2_index):
    vfg = vertex_feature.reshape(G, V, D_FEAT)
    efc = jnp.concatenate([edge_feature[:EH].reshape(G, EHP, -1),
                           edge_feature[EH:].reshape(G, EHP, -1)], axis=1)
    sl = (edge_index[0, :EH] % V).astype(jnp.int32).reshape(G, 1, EHP)
    dl = (edge_index[1, :EH] % V).astype(jnp.int32).reshape(G, 1, EHP)
    ei = jnp.concatenate([sl, dl], axis=1)
    r1i = (rev_h1_index % V).astype(jnp.int32).reshape(G, 1, A1)
    r2i = (rev_h2_index % V).astype(jnp.int32).reshape(G, A2, 3)
    r2i = r2i.transpose(0, 2, 1)
    pd = _prep_params(params)

    data_specs = [
        pl.BlockSpec((1, V, D_FEAT), lambda g: (g, 0, 0)),
        pl.BlockSpec((1, 2 * EHP, EDGE_FDIM), lambda g: (g, 0, 0)),
        pl.BlockSpec((1, 2, EHP), lambda g: (g, 0, 0)),
        pl.BlockSpec((1, 1, A1), lambda g: (g, 0, 0)),
        pl.BlockSpec((1, 3, A2), lambda g: (g, 0, 0)),
        pl.BlockSpec((V, PPER), lambda g: (0, 0)),
        pl.BlockSpec((V, PPER), lambda g: (0, 0)),
    ]
    par_specs = [pl.BlockSpec(pd[n].shape, lambda g: (0, 0)) for n in _PNAMES]

    out_shapes = [
        jax.ShapeDtypeStruct((G, 1, 1), jnp.float32),
        jax.ShapeDtypeStruct((G, PPER, 1), jnp.float32),
        jax.ShapeDtypeStruct((G, V, EHP), jnp.float32),
        jax.ShapeDtypeStruct((G, A1, 1), jnp.float32),
        jax.ShapeDtypeStruct((G, A2, 1), jnp.float32),
    ]
    out_specs = [
        pl.BlockSpec((1, 1, 1), lambda g: (g, 0, 0)),
        pl.BlockSpec((1, PPER, 1), lambda g: (g, 0, 0)),
        pl.BlockSpec((1, V, EHP), lambda g: (g, 0, 0)),
        pl.BlockSpec((1, A1, 1), lambda g: (g, 0, 0)),
        pl.BlockSpec((1, A2, 1), lambda g: (g, 0, 0)),
    ]

    stop, l1, l2, l3, l4 = pl.pallas_call(
        _body,
        grid=(G,),
        in_specs=data_specs + par_specs,
        out_specs=out_specs,
        out_shape=out_shapes,
        compiler_params=pltpu.CompilerParams(
            dimension_semantics=("parallel",)),
    )(vfg, efc, ei, r1i, r2i, _SAT, _SBT, *[pd[n] for n in _PNAMES])

    return jnp.concatenate([
        stop.reshape(G, 1),
        l1.reshape(G * PPER, 1),
        l2.reshape(G * V * EHP, 1),
        l3.reshape(G * A1, 1),
        l4.reshape(G * A2, 1),
    ], axis=0)


# speedup vs baseline: 2.2108x; 1.2836x over previous
import numpy as np
import jax
import jax.numpy as jnp
from jax import lax
from jax.experimental import pallas as pl
from jax.experimental.pallas import tpu as pltpu

G = 500
V = 20
DEG = 16
E_PER = V * DEG
E = G * E_PER
EH = E // 2
EHP = E_PER // 2
M = 128
D_FEAT = 128
EDGE_FDIM = 16
EMB = 128
HID = 128
DEPTH = 3
A1 = 100
A2 = 200
PPER = V * (V - 1) // 2
EH2 = EMB // 2

_iu0, _iu1 = np.triu_indices(V, k=1)
_SAT = (np.arange(V)[:, None] == _iu0[None, :]).astype(np.float32)
_SBT = (np.arange(V)[:, None] == _iu1[None, :]).astype(np.float32)

_PNAMES = [
    'W_edge', 'b_edge', 'W_msg', 'b_msg',
    'Wv_a', 'Wv_b', 'b_vert',
    'W_p1', 'b_p1', 'W_p2', 'b_p2',
    'ws_row', 'b_stop',
    'W1h1_0', 'W1h1_1', 'b1h1', 'Wet_h1', 'Web_h1', 'w2h1', 'b2h1',
    'W1h2_0', 'W1h2_1', 'b1h2', 'Wet_h2', 'Web_h2', 'w2h2', 'b2h2',
    'W1r1_0', 'b1r1', 'Wet_r1', 'Web_r1', 'w2r1', 'b2r1',
    'W1r2_0', 'W1r2_1', 'W1r2_2', 'b1r2', 'Wet_r2', 'Web_r2', 'w2r2', 'b2r2',
]


def _prep_params(p):
    d = {}
    d['W_edge'] = p['W_edge']
    d['b_edge'] = p['b_edge'].reshape(1, M)
    d['W_msg'] = p['W_msg']
    d['b_msg'] = p['b_msg'].reshape(1, M)
    d['Wv_a'] = p['W_vert'][:D_FEAT]
    d['Wv_b'] = p['W_vert'][D_FEAT:]
    d['b_vert'] = p['b_vert'].reshape(1, M)
    d['W_p1'] = p['W_p1']
    d['b_p1'] = p['b_p1'].reshape(1, EMB)
    d['W_p2'] = p['W_p2']
    d['b_p2'] = p['b_p2'].reshape(1, EH2)
    d['ws_row'] = p['W_stop'].T.reshape(1, EMB)
    d['b_stop'] = p['b_stop'].reshape(1, 1)
    for name, tag, nsplit in (('h1', 'h1', 2), ('h2', 'h2', 2),
                              ('rh1', 'r1', 1), ('rh2', 'r2', 3)):
        W1 = p['W1_' + name]
        for j in range(nsplit):
            d[f'W1{tag}_{j}'] = W1[j * M:(j + 1) * M]
        d[f'b1{tag}'] = p['b1_' + name].reshape(1, HID)
        We = p['We_' + name]
        d[f'Wet_{tag}'] = We[:EH2]
        d[f'Web_{tag}'] = We[EH2:]
        d[f'w2{tag}'] = p['W2_' + name].T.reshape(1, HID)
        d[f'b2{tag}'] = p['b2_' + name].reshape(1, 1)
    return d


def _dT(a, b, prec=lax.Precision.HIGHEST):
    return lax.dot_general(a, b, (((0,), (0,)), ((), ())),
                           preferred_element_type=jnp.float32,
                           precision=prec)


def _mm(a, b, prec=lax.Precision.HIGHEST):
    return jnp.dot(a, b, preferred_element_type=jnp.float32,
                   precision=prec)


_DEF = lax.Precision.DEFAULT


def _dT0(a, b):
    return _dT(a, b, _DEF)


def _mm0(a, b):
    return _mm(a, b, _DEF)


def _body(*refs):
    vf_ref, ef_ref, ei_ref, r1_ref, r2_ref, sat_ref, sbt_ref = refs[:7]
    npar = len(_PNAMES)
    P = {n: refs[7 + i][...] for i, n in enumerate(_PNAMES)}
    out_stop, out_h1, out_h2, out_r1, out_r2 = refs[7 + npar:]
    relu = jax.nn.relu

    ef = ef_ref[0]
    h0 = jnp.tanh(_mm(ef, P['W_edge']) + P['b_edge'])
    h0a = h0[:EHP]
    h0b = h0[EHP:]

    sl = ei_ref[0, 0:1, :]
    dl = ei_ref[0, 1:2, :]
    vio = lax.broadcasted_iota(jnp.int32, (V, EHP), 0)
    PT = (sl == vio).astype(jnp.float32)
    QT = (dl == vio).astype(jnp.float32)

    ha, hb = h0a, h0b
    for _ in range(DEPTH):
        agg = _mm(QT, ha) + _mm(PT, hb)
        ga = _dT(PT, agg)
        gb = _dT(QT, agg)
        ha, hb = (relu(h0a + _mm(ga - hb, P['W_msg']) + P['b_msg']),
                  relu(h0b + _mm(gb - ha, P['W_msg']) + P['b_msg']))
    aggf = _mm(QT, ha) + _mm(PT, hb)

    vf = vf_ref[0]
    vm = relu(_mm(vf, P['Wv_a']) + _mm(aggf, P['Wv_b']) + P['b_vert'])
    pre = _mm0(relu(_mm0(vm, P['W_p1']) + P['b_p1']), P['W_p2']) + P['b_p2']
    gmean = jnp.mean(pre, axis=0, keepdims=True)
    gmax = jnp.max(pre, axis=0, keepdims=True)

    stopv = (jnp.sum(gmean * P['ws_row'][:, :EH2], axis=1, keepdims=True) +
             jnp.sum(gmax * P['ws_row'][:, EH2:], axis=1, keepdims=True) +
             P['b_stop'][0, 0])
    out_stop[0] = stopv

    def head_c(tag):
        return (_mm0(gmean, P[f'Wet_{tag}']) + _mm0(gmax, P[f'Web_{tag}']) +
                P[f'b1{tag}'])

    ma = _dT0(sat_ref[...], vm)
    mb = _dT0(sbt_ref[...], vm)
    hid = relu(_mm0(jnp.minimum(ma, mb), P['W1h1_0']) +
               _mm0(jnp.maximum(ma, mb), P['W1h1_1']) + head_c('h1'))
    out_h1[0] = (jnp.sum(hid * P['w2h1'], axis=1, keepdims=True) +
                 P['b2h1'][0, 0])

    A = _mm0(vm, P['W1h2_0']) + head_c('h2')
    B = _mm0(0.5 * (ha + hb), P['W1h2_1'])
    hid3 = relu(A[:, None, :] + B[None, :, :])
    out_h2[0] = (jnp.sum(hid3 * P['w2h2'][None, :, :], axis=2) +
                 P['b2h2'][0, 0])

    vio1 = lax.broadcasted_iota(jnp.int32, (V, A1), 0)
    RT = (r1_ref[0] == vio1).astype(jnp.float32)
    f1 = _dT0(RT, vm)
    hid = relu(_mm0(f1, P['W1r1_0']) + head_c('r1'))
    out_r1[0] = (jnp.sum(hid * P['w2r1'], axis=1, keepdims=True) +
                 P['b2r1'][0, 0])

    r2 = r2_ref[0]
    vio2 = lax.broadcasted_iota(jnp.int32, (V, A2), 0)
    T0 = (r2[0:1, :] == vio2).astype(jnp.float32)
    T1 = (r2[1:2, :] == vio2).astype(jnp.float32)
    T2 = (r2[2:3, :] == vio2).astype(jnp.float32)
    m0 = _dT0(T0, vm)
    m1 = _dT0(T1, vm)
    m2 = _dT0(T2, vm)
    hid = relu(_mm0(m0, P['W1r2_0']) +
               _mm0(jnp.minimum(m1, m2), P['W1r2_1']) +
               _mm0(jnp.maximum(m1, m2), P['W1r2_2']) + head_c('r2'))
    out_r2[0] = (jnp.sum(hid * P['w2r2'], axis=1, keepdims=True) +
                 P['b2r2'][0, 0])


def kernel(vertex_feature, edge_feature, params, edge_index, rev_h1_index,
           rev_h2_index):
    vfg = vertex_feature.reshape(G, V, D_FEAT)
    efc = jnp.concatenate([edge_feature[:EH].reshape(G, EHP, -1),
                           edge_feature[EH:].reshape(G, EHP, -1)], axis=1)
    sl = (edge_index[0, :EH] % V).astype(jnp.int32).reshape(G, 1, EHP)
    dl = (edge_index[1, :EH] % V).astype(jnp.int32).reshape(G, 1, EHP)
    ei = jnp.concatenate([sl, dl], axis=1)
    r1i = (rev_h1_index % V).astype(jnp.int32).reshape(G, 1, A1)
    r2i = (rev_h2_index % V).astype(jnp.int32).reshape(G, A2, 3)
    r2i = r2i.transpose(0, 2, 1)
    pd = _prep_params(params)

    data_specs = [
        pl.BlockSpec((1, V, D_FEAT), lambda g: (g, 0, 0)),
        pl.BlockSpec((1, 2 * EHP, EDGE_FDIM), lambda g: (g, 0, 0)),
        pl.BlockSpec((1, 2, EHP), lambda g: (g, 0, 0)),
        pl.BlockSpec((1, 1, A1), lambda g: (g, 0, 0)),
        pl.BlockSpec((1, 3, A2), lambda g: (g, 0, 0)),
        pl.BlockSpec((V, PPER), lambda g: (0, 0)),
        pl.BlockSpec((V, PPER), lambda g: (0, 0)),
    ]
    par_specs = [pl.BlockSpec(pd[n].shape, lambda g: (0, 0)) for n in _PNAMES]

    out_shapes = [
        jax.ShapeDtypeStruct((G, 1, 1), jnp.float32),
        jax.ShapeDtypeStruct((G, PPER, 1), jnp.float32),
        jax.ShapeDtypeStruct((G, V, EHP), jnp.float32),
        jax.ShapeDtypeStruct((G, A1, 1), jnp.float32),
        jax.ShapeDtypeStruct((G, A2, 1), jnp.float32),
    ]
    out_specs = [
        pl.BlockSpec((1, 1, 1), lambda g: (g, 0, 0)),
        pl.BlockSpec((1, PPER, 1), lambda g: (g, 0, 0)),
        pl.BlockSpec((1, V, EHP), lambda g: (g, 0, 0)),
        pl.BlockSpec((1, A1, 1), lambda g: (g, 0, 0)),
        pl.BlockSpec((1, A2, 1), lambda g: (g, 0, 0)),
    ]

    stop, l1, l2, l3, l4 = pl.pallas_call(
        _body,
        grid=(G,),
        in_specs=data_specs + par_specs,
        out_specs=out_specs,
        out_shape=out_shapes,
        compiler_params=pltpu.CompilerParams(
            dimension_semantics=("parallel",)),
    )(vfg, efc, ei, r1i, r2i, _SAT, _SBT, *[pd[n] for n in _PNAMES])

    return jnp.concatenate([
        stop.reshape(G, 1),
        l1.reshape(G * PPER, 1),
        l2.reshape(G * V * EHP, 1),
        l3.reshape(G * A1, 1),
        l4.reshape(G * A2, 1),
    ], axis=0)


# speedup vs baseline: 2.3696x; 1.0718x over previous
import numpy as np
import jax
import jax.numpy as jnp
from jax import lax
from jax.experimental import pallas as pl
from jax.experimental.pallas import tpu as pltpu

G = 500
V = 20
DEG = 16
E_PER = V * DEG
E = G * E_PER
EH = E // 2
EHP = E_PER // 2
M = 128
D_FEAT = 128
EDGE_FDIM = 16
EMB = 128
HID = 128
DEPTH = 3
A1 = 100
A2 = 200
PPER = V * (V - 1) // 2
GB = 4
EH2 = EMB // 2

_iu0, _iu1 = np.triu_indices(V, k=1)
_SAT = (np.arange(V)[:, None] == _iu0[None, :]).astype(np.float32)
_SBT = (np.arange(V)[:, None] == _iu1[None, :]).astype(np.float32)

_PNAMES = [
    'W_edge', 'b_edge', 'W_msg', 'b_msg',
    'Wv_a', 'Wv_b', 'b_vert',
    'W_p1', 'b_p1', 'W_p2', 'b_p2',
    'ws_row', 'b_stop',
    'W1h1_0', 'W1h1_1', 'b1h1', 'Wet_h1', 'Web_h1', 'w2h1', 'b2h1',
    'W1h2_0', 'W1h2_1', 'b1h2', 'Wet_h2', 'Web_h2', 'w2h2', 'b2h2',
    'W1r1_0', 'b1r1', 'Wet_r1', 'Web_r1', 'w2r1', 'b2r1',
    'W1r2_0', 'W1r2_1', 'W1r2_2', 'b1r2', 'Wet_r2', 'Web_r2', 'w2r2', 'b2r2',
]


def _prep_params(p):
    d = {}
    d['W_edge'] = p['W_edge']
    d['b_edge'] = p['b_edge'].reshape(1, M)
    d['W_msg'] = p['W_msg']
    d['b_msg'] = p['b_msg'].reshape(1, M)
    d['Wv_a'] = p['W_vert'][:D_FEAT]
    d['Wv_b'] = p['W_vert'][D_FEAT:]
    d['b_vert'] = p['b_vert'].reshape(1, M)
    d['W_p1'] = p['W_p1']
    d['b_p1'] = p['b_p1'].reshape(1, EMB)
    d['W_p2'] = p['W_p2']
    d['b_p2'] = p['b_p2'].reshape(1, EH2)
    d['ws_row'] = p['W_stop'].T.reshape(1, EMB)
    d['b_stop'] = p['b_stop'].reshape(1, 1)
    for name, tag, nsplit in (('h1', 'h1', 2), ('h2', 'h2', 2),
                              ('rh1', 'r1', 1), ('rh2', 'r2', 3)):
        W1 = p['W1_' + name]
        for j in range(nsplit):
            d[f'W1{tag}_{j}'] = W1[j * M:(j + 1) * M]
        d[f'b1{tag}'] = p['b1_' + name].reshape(1, HID)
        We = p['We_' + name]
        d[f'Wet_{tag}'] = We[:EH2]
        d[f'Web_{tag}'] = We[EH2:]
        d[f'w2{tag}'] = p['W2_' + name].T.reshape(1, HID)
        d[f'b2{tag}'] = p['b2_' + name].reshape(1, 1)
    return d


def _dT(a, b, prec=lax.Precision.HIGHEST):
    return lax.dot_general(a, b, (((0,), (0,)), ((), ())),
                           preferred_element_type=jnp.float32,
                           precision=prec)


def _mm(a, b, prec=lax.Precision.HIGHEST):
    return jnp.dot(a, b, preferred_element_type=jnp.float32,
                   precision=prec)


_DEF = lax.Precision.DEFAULT


def _dT0(a, b):
    return _dT(a, b, _DEF)


def _mm0(a, b):
    return _mm(a, b, _DEF)


def _body(*refs):
    vf_ref, ef_ref, ei_ref, r1_ref, r2_ref, sat_ref, sbt_ref = refs[:7]
    npar = len(_PNAMES)
    P = {n: refs[7 + i][...] for i, n in enumerate(_PNAMES)}
    out_stop, out_h1, out_h2, out_r1, out_r2 = refs[7 + npar:]
    relu = jax.nn.relu
    for b in range(GB):
        _graph(b, vf_ref, ef_ref, ei_ref, r1_ref, r2_ref, sat_ref, sbt_ref, P,
               out_stop, out_h1, out_h2, out_r1, out_r2)


def _graph(b, vf_ref, ef_ref, ei_ref, r1_ref, r2_ref, sat_ref, sbt_ref, P,
           out_stop, out_h1, out_h2, out_r1, out_r2):
    relu = jax.nn.relu

    ef = ef_ref[b]
    h0 = jnp.tanh(_mm(ef, P['W_edge']) + P['b_edge'])
    h0a = h0[:EHP]
    h0b = h0[EHP:]

    sl = ei_ref[b, 0:1, :]
    dl = ei_ref[b, 1:2, :]
    vio = lax.broadcasted_iota(jnp.int32, (V, EHP), 0)
    PT = (sl == vio).astype(jnp.float32)
    QT = (dl == vio).astype(jnp.float32)

    ha, hb = h0a, h0b
    for _ in range(DEPTH):
        agg = _mm(QT, ha) + _mm(PT, hb)
        ga = _dT(PT, agg)
        gb = _dT(QT, agg)
        ha, hb = (relu(h0a + _mm(ga - hb, P['W_msg']) + P['b_msg']),
                  relu(h0b + _mm(gb - ha, P['W_msg']) + P['b_msg']))
    aggf = _mm(QT, ha) + _mm(PT, hb)

    vf = vf_ref[b]
    vm = relu(_mm(vf, P['Wv_a']) + _mm(aggf, P['Wv_b']) + P['b_vert'])
    pre = _mm0(relu(_mm0(vm, P['W_p1']) + P['b_p1']), P['W_p2']) + P['b_p2']
    gmean = jnp.mean(pre, axis=0, keepdims=True)
    gmax = jnp.max(pre, axis=0, keepdims=True)

    stopv = (jnp.sum(gmean * P['ws_row'][:, :EH2], axis=1, keepdims=True) +
             jnp.sum(gmax * P['ws_row'][:, EH2:], axis=1, keepdims=True) +
             P['b_stop'][0, 0])
    out_stop[b] = stopv

    def head_c(tag):
        return (_mm0(gmean, P[f'Wet_{tag}']) + _mm0(gmax, P[f'Web_{tag}']) +
                P[f'b1{tag}'])

    ma = _dT0(sat_ref[...], vm)
    mb = _dT0(sbt_ref[...], vm)
    hid = relu(_mm0(jnp.minimum(ma, mb), P['W1h1_0']) +
               _mm0(jnp.maximum(ma, mb), P['W1h1_1']) + head_c('h1'))
    out_h1[b] = (jnp.sum(hid * P['w2h1'], axis=1, keepdims=True) +
                 P['b2h1'][0, 0])

    A = _mm0(vm, P['W1h2_0']) + head_c('h2')
    B = _mm0(0.5 * (ha + hb), P['W1h2_1'])
    hid3 = relu(A[:, None, :] + B[None, :, :])
    out_h2[b] = (jnp.sum(hid3 * P['w2h2'][None, :, :], axis=2) +
                 P['b2h2'][0, 0])

    vio1 = lax.broadcasted_iota(jnp.int32, (V, A1), 0)
    RT = (r1_ref[b] == vio1).astype(jnp.float32)
    f1 = _dT0(RT, vm)
    hid = relu(_mm0(f1, P['W1r1_0']) + head_c('r1'))
    out_r1[b] = (jnp.sum(hid * P['w2r1'], axis=1, keepdims=True) +
                 P['b2r1'][0, 0])

    r2 = r2_ref[b]
    vio2 = lax.broadcasted_iota(jnp.int32, (V, A2), 0)
    T0 = (r2[0:1, :] == vio2).astype(jnp.float32)
    T1 = (r2[1:2, :] == vio2).astype(jnp.float32)
    T2 = (r2[2:3, :] == vio2).astype(jnp.float32)
    m0 = _dT0(T0, vm)
    m1 = _dT0(T1, vm)
    m2 = _dT0(T2, vm)
    hid = relu(_mm0(m0, P['W1r2_0']) +
               _mm0(jnp.minimum(m1, m2), P['W1r2_1']) +
               _mm0(jnp.maximum(m1, m2), P['W1r2_2']) + head_c('r2'))
    out_r2[b] = (jnp.sum(hid * P['w2r2'], axis=1, keepdims=True) +
                 P['b2r2'][0, 0])


def kernel(vertex_feature, edge_feature, params, edge_index, rev_h1_index,
           rev_h2_index):
    vfg = vertex_feature.reshape(G, V, D_FEAT)
    efc = jnp.concatenate([edge_feature[:EH].reshape(G, EHP, -1),
                           edge_feature[EH:].reshape(G, EHP, -1)], axis=1)
    sl = (edge_index[0, :EH] % V).astype(jnp.int32).reshape(G, 1, EHP)
    dl = (edge_index[1, :EH] % V).astype(jnp.int32).reshape(G, 1, EHP)
    ei = jnp.concatenate([sl, dl], axis=1)
    r1i = (rev_h1_index % V).astype(jnp.int32).reshape(G, 1, A1)
    r2i = (rev_h2_index % V).astype(jnp.int32).reshape(G, A2, 3)
    r2i = r2i.transpose(0, 2, 1)
    pd = _prep_params(params)

    data_specs = [
        pl.BlockSpec((GB, V, D_FEAT), lambda g: (g, 0, 0)),
        pl.BlockSpec((GB, 2 * EHP, EDGE_FDIM), lambda g: (g, 0, 0)),
        pl.BlockSpec((GB, 2, EHP), lambda g: (g, 0, 0)),
        pl.BlockSpec((GB, 1, A1), lambda g: (g, 0, 0)),
        pl.BlockSpec((GB, 3, A2), lambda g: (g, 0, 0)),
        pl.BlockSpec((V, PPER), lambda g: (0, 0)),
        pl.BlockSpec((V, PPER), lambda g: (0, 0)),
    ]
    par_specs = [pl.BlockSpec(pd[n].shape, lambda g: (0, 0)) for n in _PNAMES]

    out_shapes = [
        jax.ShapeDtypeStruct((G, 1, 1), jnp.float32),
        jax.ShapeDtypeStruct((G, PPER, 1), jnp.float32),
        jax.ShapeDtypeStruct((G, V, EHP), jnp.float32),
        jax.ShapeDtypeStruct((G, A1, 1), jnp.float32),
        jax.ShapeDtypeStruct((G, A2, 1), jnp.float32),
    ]
    out_specs = [
        pl.BlockSpec((GB, 1, 1), lambda g: (g, 0, 0)),
        pl.BlockSpec((GB, PPER, 1), lambda g: (g, 0, 0)),
        pl.BlockSpec((GB, V, EHP), lambda g: (g, 0, 0)),
        pl.BlockSpec((GB, A1, 1), lambda g: (g, 0, 0)),
        pl.BlockSpec((GB, A2, 1), lambda g: (g, 0, 0)),
    ]

    stop, l1, l2, l3, l4 = pl.pallas_call(
        _body,
        grid=(G // GB,),
        in_specs=data_specs + par_specs,
        out_specs=out_specs,
        out_shape=out_shapes,
        compiler_params=pltpu.CompilerParams(
            dimension_semantics=("parallel",)),
    )(vfg, efc, ei, r1i, r2i, _SAT, _SBT, *[pd[n] for n in _PNAMES])

    return jnp.concatenate([
        stop.reshape(G, 1),
        l1.reshape(G * PPER, 1),
        l2.reshape(G * V * EHP, 1),
        l3.reshape(G * A1, 1),
        l4.reshape(G * A2, 1),
    ], axis=0)
